# baseline (speedup 1.0000x reference)
import jax
import jax.numpy as jnp
from jax import lax
from jax.experimental import pallas as pl
from jax.experimental.pallas import tpu as pltpu
from jax.experimental.pallas import tpu_sc as plsc

_NC = 2
_NS = 16
_NW = _NC * _NS
_LANES = 16
_OUT_DIM = 1024
_GPR = _OUT_DIM // _LANES
_CHUNK = 16
_CROWS = _CHUNK * _GPR
_NBUF = 3
_TOTAL_ROWS = 4 * 4096
_ROWS_PER_W = _TOTAL_ROWS // _NW
_NCHUNKS = _ROWS_PER_W // _CHUNK
_TBL_ROWS = 4096


def _sc_body(x_hbm, t_hbm, o_hbm, xbuf, tbuf, in_sems, t_sems, out_sems):
    cid = lax.axis_index("c")
    sid = lax.axis_index("s")
    wid = sid * _NC + cid
    base = wid * _ROWS_PER_W * _GPR
    tbase = (wid * _ROWS_PER_W % _TBL_ROWS) * _GPR

    def in_copy(k):
        slot = k % _NBUF
        return pltpu.make_async_copy(
            x_hbm.at[pl.ds(base + k * _CROWS, _CROWS), :],
            xbuf.at[slot],
            in_sems.at[slot],
        )

    def t_copy(k):
        slot = k % _NBUF
        return pltpu.make_async_copy(
            t_hbm.at[pl.ds(tbase + k * _CROWS, _CROWS), :],
            tbuf.at[slot],
            t_sems.at[slot],
        )

    def out_copy(k):
        slot = k % _NBUF
        return pltpu.make_async_copy(
            xbuf.at[slot],
            o_hbm.at[pl.ds(base + k * _CROWS, _CROWS), :],
            out_sems.at[slot],
        )

    for k in range(_NBUF):
        in_copy(k).start()
        t_copy(k).start()

    for k in range(_NCHUNKS):
        slot = k % _NBUF
        in_copy(k).wait()
        t_copy(k).wait()

        def body(j, _):
            xbuf[slot, j] = xbuf[slot, j] + tbuf[slot, j]
            return 0

        lax.fori_loop(0, _CROWS, body, 0, unroll=8)
        out_copy(k).start()
        nxt = k + _NBUF
        if nxt < _NCHUNKS:
            out_copy(k).wait()
            in_copy(nxt).start()
            t_copy(nxt).start()

    for k in range(_NCHUNKS - _NBUF, _NCHUNKS):
        out_copy(k).wait()


def kernel(inputs, pos_table):
    batch, seq_len, out_dim = inputs.shape
    flat = inputs.reshape(batch * seq_len * _GPR, _LANES)
    tbl = pos_table.reshape(seq_len * _GPR, _LANES)
    mesh = plsc.VectorSubcoreMesh(core_axis_name="c", subcore_axis_name="s")
    out = pl.kernel(
        _sc_body,
        out_type=jax.ShapeDtypeStruct(flat.shape, flat.dtype),
        mesh=mesh,
        compiler_params=pltpu.CompilerParams(use_tc_tiling_on_sc=False),
        scratch_types=[
            pltpu.VMEM((_NBUF, _CROWS, _LANES), jnp.float32),
            pltpu.VMEM((_NBUF, _CROWS, _LANES), jnp.float32),
            pltpu.SemaphoreType.DMA((_NBUF,)),
            pltpu.SemaphoreType.DMA((_NBUF,)),
            pltpu.SemaphoreType.DMA((_NBUF,)),
        ],
    )(flat, tbl)
    return out.reshape(batch, seq_len, out_dim)
